# baseline (speedup 1.0000x reference)
import functools

import jax
import jax.numpy as jnp
from jax.experimental import pallas as pl
from jax.experimental.pallas import tpu as pltpu

_PH = (768, 1536)


def _level_body(resid_ref, w_ref, b2_ref, a2_ref, idx_ref, q_ref, *, K):
    resid = resid_ref[...]
    w = w_ref[...]
    rb = resid.astype(jnp.bfloat16)
    wb = w.astype(jnp.bfloat16)
    conv = jax.lax.dot_general(
        rb, wb, (((1,), (1,)), ((), ())),
        preferred_element_type=jnp.float32)
    a2 = a2_ref[...]
    b2 = b2_ref[...]
    s = jnp.maximum((a2 + b2) - 2.0 * conv, 0.0)
    dd = s * jax.lax.rsqrt(s)
    R = resid.shape[0]
    iota = jax.lax.broadcasted_iota(jnp.int32, (R, K), 1)
    acc_v = None
    acc_i = None
    for lo, hi in ((0, _PH[0]), (_PH[0], _PH[1]), (_PH[1], K)):
        seg = dd[:, lo:hi]
        m = jnp.min(seg, axis=1, keepdims=True)
        si = jnp.min(jnp.where(seg == m, iota[:, lo:hi], K),
                     axis=1, keepdims=True)
        mb = m.astype(jnp.bfloat16).astype(jnp.float32)
        if acc_v is None:
            acc_v, acc_i = mb, si
        else:
            upd = m < acc_v
            acc_i = jnp.where(upd, si, acc_i)
            acc_v = jnp.where(upd, mb, acc_v)
    idx_ref[...] = acc_i
    onehot = (iota == acc_i).astype(jnp.float32)
    q_ref[...] = jax.lax.dot_general(
        onehot, w, (((1,), (0,)), ((), ())),
        precision=jax.lax.Precision.HIGHEST,
        preferred_element_type=jnp.float32)


def _level(resid, w, b2, a2, *, rows):
    N, D = resid.shape
    K = w.shape[0]
    grid = (N // rows,)
    idx, q = pl.pallas_call(
        functools.partial(_level_body, K=K),
        grid=grid,
        in_specs=[
            pl.BlockSpec((rows, D), lambda i: (i, 0)),
            pl.BlockSpec((K, D), lambda i: (0, 0)),
            pl.BlockSpec((1, K), lambda i: (0, 0)),
            pl.BlockSpec((rows, 1), lambda i: (i, 0)),
        ],
        out_specs=[
            pl.BlockSpec((rows, 1), lambda i: (i, 0)),
            pl.BlockSpec((rows, D), lambda i: (i, 0)),
        ],
        out_shape=[
            jax.ShapeDtypeStruct((N, 1), jnp.int32),
            jax.ShapeDtypeStruct((N, D), jnp.float32),
        ],
        compiler_params=pltpu.CompilerParams(
            dimension_semantics=("arbitrary",)),
    )(resid, w, b2, a2)
    return idx, q


def kernel(embeddings, codebooks):
    B_, T_, D_ = embeddings.shape
    n_cb, K, _ = codebooks.shape
    N = B_ * T_
    rows = 600
    quantized = jnp.zeros_like(embeddings)
    residual = embeddings
    total_loss = jnp.float32(0.0)
    all_codes = []
    for l in range(n_cb):
        w = codebooks[l]
        b2 = jnp.sum(w * w, axis=1).reshape(1, K)
        a2 = jnp.sum(residual * residual, axis=2).reshape(N, 1)
        idx, q = _level(residual.reshape(N, D_), w, b2, a2, rows=rows)
        all_codes.append(idx.reshape(B_, T_))
        q = q.reshape(B_, T_, D_)
        quantized = quantized + q
        total_loss = total_loss + jnp.mean(
            (jax.lax.stop_gradient(residual) - q) ** 2)
        residual = residual - jax.lax.stop_gradient(q)
    codes = jnp.stack(all_codes, axis=-1)
    quantized = embeddings + jax.lax.stop_gradient(quantized - embeddings)
    return codes, quantized, total_loss / n_cb


# speedup vs baseline: 1.5001x; 1.5001x over previous
import functools

import jax
import jax.numpy as jnp
from jax import lax
from jax.experimental import pallas as pl
from jax.experimental.pallas import tpu as pltpu
from jax.experimental.pallas import tpu_sc as plsc

_PH = (768, 1536)


def _level_body(resid_ref, w_ref, b2_ref, a2_ref, idx_ref, *, K):
    resid = resid_ref[...]
    w = w_ref[...]
    rb = resid.astype(jnp.bfloat16)
    wb = w.astype(jnp.bfloat16)
    conv = jax.lax.dot_general(
        rb, wb, (((1,), (1,)), ((), ())),
        preferred_element_type=jnp.float32)
    a2 = a2_ref[...]
    b2 = b2_ref[...]
    s = jnp.maximum((a2 + b2) - 2.0 * conv, 0.0)
    dd = s * jax.lax.rsqrt(s)
    R = resid.shape[0]
    iota = jax.lax.broadcasted_iota(jnp.int32, (R, K), 1)
    acc_v = None
    acc_i = None
    for lo, hi in ((0, _PH[0]), (_PH[0], _PH[1]), (_PH[1], K)):
        seg = dd[:, lo:hi]
        m = jnp.min(seg, axis=1, keepdims=True)
        si = jnp.min(jnp.where(seg == m, iota[:, lo:hi], K),
                     axis=1, keepdims=True)
        mb = m.astype(jnp.bfloat16).astype(jnp.float32)
        if acc_v is None:
            acc_v, acc_i = mb, si
        else:
            upd = m < acc_v
            acc_i = jnp.where(upd, si, acc_i)
            acc_v = jnp.where(upd, mb, acc_v)
    idx_ref[...] = acc_i


def _level_indices(resid, w, b2, a2, *, rows):
    N, D = resid.shape
    K = w.shape[0]
    grid = (N // rows,)
    idx = pl.pallas_call(
        functools.partial(_level_body, K=K),
        grid=grid,
        in_specs=[
            pl.BlockSpec((rows, D), lambda i: (i, 0)),
            pl.BlockSpec((K, D), lambda i: (0, 0)),
            pl.BlockSpec((1, K), lambda i: (0, 0)),
            pl.BlockSpec((rows, 1), lambda i: (i, 0)),
        ],
        out_specs=pl.BlockSpec((rows, 1), lambda i: (i, 0)),
        out_shape=jax.ShapeDtypeStruct((N, 1), jnp.int32),
        compiler_params=pltpu.CompilerParams(
            dimension_semantics=("arbitrary",)),
    )(resid, w, b2, a2)
    return idx


def _make_sc_gather(V, D, B):
    info = plsc.get_sparse_core_info()
    NC, NS = info.num_cores, info.num_subcores
    NW = NC * NS
    b_per_w = B // NW
    n_chunks = 4
    ch = b_per_w // n_chunks
    mesh = plsc.VectorSubcoreMesh(core_axis_name="c", subcore_axis_name="s")

    @functools.partial(
        pl.kernel, mesh=mesh,
        out_type=jax.ShapeDtypeStruct((B, D), jnp.float32),
        scratch_types=[
            pltpu.VMEM((ch,), jnp.int32),
            pltpu.VMEM((ch, D), jnp.float32),
            pltpu.SemaphoreType.DMA,
        ],
    )
    def k(table_hbm, idx_hbm, out_hbm, idx_v, rows_v, sem):
        wid = lax.axis_index("s") * NC + lax.axis_index("c")
        base = wid * b_per_w
        for c in range(n_chunks):
            off = base + c * ch
            pltpu.sync_copy(idx_hbm.at[pl.ds(off, ch)], idx_v)
            pltpu.async_copy(table_hbm.at[idx_v], rows_v, sem).wait()
            pltpu.sync_copy(rows_v, out_hbm.at[pl.ds(off, ch)])

    return k


def kernel(embeddings, codebooks):
    B_, T_, D_ = embeddings.shape
    n_cb, K, _ = codebooks.shape
    N = B_ * T_
    rows = 600
    NPAD = ((N + 1023) // 1024) * 1024
    gather = _make_sc_gather(K, D_, NPAD)
    quantized = jnp.zeros_like(embeddings)
    residual = embeddings
    total_loss = jnp.float32(0.0)
    all_codes = []
    for l in range(n_cb):
        w = codebooks[l]
        b2 = jnp.sum(w * w, axis=1).reshape(1, K)
        a2 = jnp.sum(residual * residual, axis=2).reshape(N, 1)
        idx = _level_indices(residual.reshape(N, D_), w, b2, a2, rows=rows)
        idx1 = idx.reshape(N)
        all_codes.append(idx1.reshape(B_, T_))
        idxp = jnp.pad(idx1, (0, NPAD - N))
        q = gather(w, idxp)[:N].reshape(B_, T_, D_)
        quantized = quantized + q
        total_loss = total_loss + jnp.mean(
            (jax.lax.stop_gradient(residual) - q) ** 2)
        residual = residual - jax.lax.stop_gradient(q)
    codes = jnp.stack(all_codes, axis=-1)
    quantized = embeddings + jax.lax.stop_gradient(quantized - embeddings)
    return codes, quantized, total_loss / n_cb
